# baseline (speedup 1.0000x reference)
import functools

import numpy as np

import jax
import jax.numpy as jnp
from jax import lax
from jax.experimental import pallas as pl
from jax.experimental.pallas import tpu as pltpu
from jax.experimental.pallas import tpu_sc as plsc

EMBED_DIM = 64
SEQ = 200
BATCH = 4096
NUM_WORKERS = 32
ROWS_PER_W = BATCH // NUM_WORKERS
LANES = 16
DCH = EMBED_DIM // LANES
CHUNK0 = 128
CHUNK1 = SEQ - CHUNK0


NBUF = 4
HALF = 500000
TW = 1024


def _sc_pool(text, table):
    mesh = plsc.VectorSubcoreMesh(core_axis_name="c", subcore_axis_name="s")

    @functools.partial(
        pl.kernel,
        out_type=jax.ShapeDtypeStruct((BATCH, EMBED_DIM), jnp.float32),
        mesh=mesh,
        compiler_params=pltpu.CompilerParams(use_tc_tiling_on_sc=False),
        scratch_types=[
            pltpu.VMEM((ROWS_PER_W, SEQ), jnp.int32),
            pltpu.VMEM((NBUF, SEQ, EMBED_DIM), jnp.float32),
            pltpu.VMEM((ROWS_PER_W, EMBED_DIM), jnp.float32),
            [pltpu.SemaphoreType.DMA] * NBUF,
        ],
    )
    def k(text_hbm, table_hbm, out_hbm, idx_v, rows_v, out_v, sems):
        wid = lax.axis_index("s") * 2 + lax.axis_index("c")
        base = wid * ROWS_PER_W
        pltpu.sync_copy(text_hbm.at[pl.ds(base, ROWS_PER_W)], idx_v)

        scale = jnp.float32(1.0 / SEQ)

        def fire(r, s):
            pltpu.async_copy(
                table_hbm.at[idx_v.at[r, pl.ds(0, CHUNK0)]],
                rows_v.at[s, pl.ds(0, CHUNK0)], sems[s])
            pltpu.async_copy(
                table_hbm.at[idx_v.at[r, pl.ds(CHUNK0, CHUNK1)]],
                rows_v.at[s, pl.ds(CHUNK0, CHUNK1)], sems[s])

        def drain(s):
            pltpu.make_async_copy(
                table_hbm.at[pl.ds(0, SEQ)], rows_v.at[s], sems[s]).wait()

        def reduce_slot(s, r):
            def seq_body(j, acc):
                return tuple(
                    acc[k_] + rows_v[s, j, pl.ds(k_ * LANES, LANES)]
                    for k_ in range(DCH))

            zeros = tuple(jnp.zeros((LANES,), jnp.float32) for _ in range(DCH))
            acc = lax.fori_loop(0, SEQ, seq_body, zeros, unroll=2)
            for k_ in range(DCH):
                out_v[r, pl.ds(k_ * LANES, LANES)] = acc[k_] * scale

        for s in range(NBUF):
            fire(s, s)

        def group_body(g, _):
            rg = g * NBUF
            for s in range(NBUF):
                drain(s)
                reduce_slot(s, rg + s)
                fire(rg + s + NBUF, s)
            return 0

        lax.fori_loop(0, ROWS_PER_W // NBUF - 1, group_body, 0)
        rg = ROWS_PER_W - NBUF
        for s in range(NBUF):
            drain(s)
            reduce_slot(s, rg + s)

        pltpu.sync_copy(out_v, out_hbm.at[pl.ds(base, ROWS_PER_W)])

    return k(text, table)


def _tc_head(pooled, W, b):
    def body(p_ref, w_ref, b_ref, o_ref):
        p = p_ref[...]
        w = w_ref[...]
        logits = lax.dot_general(
            p, w, dimension_numbers=(((1,), (1,)), ((), ())),
            preferred_element_type=jnp.float32)
        logits = logits + b_ref[...]
        m = jnp.max(logits, axis=1, keepdims=True)
        lse = m + jnp.log(jnp.sum(jnp.exp(logits - m), axis=1, keepdims=True))
        o_ref[...] = logits - lse

    return pl.pallas_call(
        body,
        out_shape=jax.ShapeDtypeStruct((BATCH, 2), jnp.float32),
    )(pooled, W, b.reshape(1, 2))


def _tc_repack(table):
    table_t = table.T

    def body(x_ref, o_ref2):
        xt = x_ref[...].T
        y3 = xt.reshape(TW // 2, 2, EMBED_DIM)
        o_ref2[...] = jnp.concatenate([y3[:, 0, :], y3[:, 1, :]], axis=1)

    n = -(-(2 * HALF) // TW)
    repacked = pl.pallas_call(
        body,
        grid=(n,),
        in_specs=[pl.BlockSpec((EMBED_DIM, TW), lambda i: (0, i))],
        out_specs=pl.BlockSpec((TW // 2, 2 * EMBED_DIM), lambda i: (i, 0)),
        out_shape=jax.ShapeDtypeStruct((HALF, 2 * EMBED_DIM), jnp.float32),
    )(table_t)
    return repacked.reshape(2 * HALF, EMBED_DIM)


def kernel(text, table, W, b):
    pooled = _sc_pool(text.astype(jnp.int32), _tc_repack(table))
    return _tc_head(pooled, W, b)


# speedup vs baseline: 1.2630x; 1.2630x over previous
import functools

import jax
import jax.numpy as jnp
from jax import lax
from jax.experimental import pallas as pl
from jax.experimental.pallas import tpu as pltpu
from jax.experimental.pallas import tpu_sc as plsc

EMBED_DIM = 64
SEQ = 200
BATCH = 4096
NUM_WORKERS = 32
ROWS_PER_W = BATCH // NUM_WORKERS
LANES = 16
DCH = EMBED_DIM // LANES
CHUNK0 = 128
CHUNK1 = SEQ - CHUNK0


NBUF = 4


def _sc_pool(text, table):
    mesh = plsc.VectorSubcoreMesh(core_axis_name="c", subcore_axis_name="s")

    @functools.partial(
        pl.kernel,
        out_type=jax.ShapeDtypeStruct((BATCH, EMBED_DIM), jnp.float32),
        mesh=mesh,
        compiler_params=pltpu.CompilerParams(use_tc_tiling_on_sc=False),
        scratch_types=[
            pltpu.VMEM((ROWS_PER_W, SEQ), jnp.int32),
            pltpu.VMEM((NBUF, SEQ, EMBED_DIM), jnp.float32),
            pltpu.VMEM((ROWS_PER_W, EMBED_DIM), jnp.float32),
            [pltpu.SemaphoreType.DMA] * NBUF,
        ],
    )
    def k(text_hbm, table_hbm, out_hbm, idx_v, rows_v, out_v, sems):
        wid = lax.axis_index("s") * 2 + lax.axis_index("c")
        base = wid * ROWS_PER_W
        pltpu.sync_copy(text_hbm.at[pl.ds(base, ROWS_PER_W)], idx_v)

        scale = jnp.float32(1.0 / SEQ)

        def fire(r, s):
            pltpu.async_copy(
                table_hbm.at[idx_v.at[r, pl.ds(0, CHUNK0)]],
                rows_v.at[s, pl.ds(0, CHUNK0)], sems[s])
            pltpu.async_copy(
                table_hbm.at[idx_v.at[r, pl.ds(CHUNK0, CHUNK1)]],
                rows_v.at[s, pl.ds(CHUNK0, CHUNK1)], sems[s])

        def drain(s):
            pltpu.make_async_copy(
                table_hbm.at[pl.ds(0, SEQ)], rows_v.at[s], sems[s]).wait()

        def reduce_slot(s, r):
            def seq_body(j, acc):
                return tuple(
                    acc[k_] + rows_v[s, j, pl.ds(k_ * LANES, LANES)]
                    for k_ in range(DCH))

            zeros = tuple(jnp.zeros((LANES,), jnp.float32) for _ in range(DCH))
            acc = lax.fori_loop(0, SEQ, seq_body, zeros, unroll=2)
            for k_ in range(DCH):
                out_v[r, pl.ds(k_ * LANES, LANES)] = acc[k_] * scale

        for s in range(NBUF):
            fire(s, s)

        def group_body(g, _):
            rg = g * NBUF
            for s in range(NBUF):
                drain(s)
                reduce_slot(s, rg + s)
                fire(rg + s + NBUF, s)
            return 0

        lax.fori_loop(0, ROWS_PER_W // NBUF - 1, group_body, 0)
        rg = ROWS_PER_W - NBUF
        for s in range(NBUF):
            drain(s)
            reduce_slot(s, rg + s)

        pltpu.sync_copy(out_v, out_hbm.at[pl.ds(base, ROWS_PER_W)])

    return k(text, table)


def _tc_head(pooled, W, b):
    def body(p_ref, w_ref, b_ref, o_ref):
        p = p_ref[...]
        w = w_ref[...]
        logits = lax.dot_general(
            p, w, dimension_numbers=(((1,), (1,)), ((), ())),
            preferred_element_type=jnp.float32)
        logits = logits + b_ref[...]
        m = jnp.max(logits, axis=1, keepdims=True)
        lse = m + jnp.log(jnp.sum(jnp.exp(logits - m), axis=1, keepdims=True))
        o_ref[...] = logits - lse

    return pl.pallas_call(
        body,
        out_shape=jax.ShapeDtypeStruct((BATCH, 2), jnp.float32),
    )(pooled, W, b.reshape(1, 2))


def kernel(text, table, W, b):
    pooled = _sc_pool(text.astype(jnp.int32), table)
    return _tc_head(pooled, W, b)
